# baseline (speedup 1.0000x reference)
import functools

import jax
import jax.numpy as jnp
from jax import lax
from jax.experimental import pallas as pl
from jax.experimental.pallas import tpu as pltpu
from jax.experimental.pallas import tpu_sc as plsc

_NC, _NS = 2, 16
_NW = _NC * _NS
_SEQ = 8192
_DIM = 2048
_TC_ROWS = 5120
_SC_ROWS = _SEQ - _TC_ROWS
_ROWS_PER_W = _SC_ROWS // _NW
_CHUNK = 16
_NCHUNKS = _ROWS_PER_W // _CHUNK
_NBUF = 2


def _sc_body(w_hbm, out_hbm, buf, sems):
    wid = lax.axis_index("s") * _NC + lax.axis_index("c")
    base = wid * _ROWS_PER_W

    def cp_in(c):
        return pltpu.make_async_copy(
            w_hbm.at[pl.ds(base + c * _CHUNK, _CHUNK), :],
            buf.at[c % _NBUF],
            sems.at[c % _NBUF],
        )

    def cp_out(c):
        return pltpu.make_async_copy(
            buf.at[c % _NBUF],
            out_hbm.at[pl.ds(base + c * _CHUNK, _CHUNK), :],
            sems.at[_NBUF + c % _NBUF],
        )

    for c in range(_NBUF):
        cp_in(c).start()
    for c in range(_NCHUNKS):
        cp_in(c).wait()
        cp_out(c).start()
        if c + _NBUF < _NCHUNKS:
            cp_out(c).wait()
            cp_in(c + _NBUF).start()
    for c in range(max(0, _NCHUNKS - _NBUF), _NCHUNKS):
        cp_out(c).wait()


def _tc_body(w_ref, o_ref):
    o_ref[...] = w_ref[...]


def kernel(x, emb_weight):
    seq_len = x.shape[1]
    dim = emb_weight.shape[1]

    rows_per_block = 1024
    tc_out = pl.pallas_call(
        _tc_body,
        grid=(_TC_ROWS // rows_per_block,),
        in_specs=[pl.BlockSpec((rows_per_block, dim), lambda i: (i, 0))],
        out_specs=pl.BlockSpec((rows_per_block, dim), lambda i: (i, 0)),
        out_shape=jax.ShapeDtypeStruct((_TC_ROWS, dim), emb_weight.dtype),
    )(emb_weight[:_TC_ROWS])

    mesh = plsc.VectorSubcoreMesh(core_axis_name="c", subcore_axis_name="s")
    sc_k = functools.partial(
        pl.kernel,
        mesh=mesh,
        out_type=jax.ShapeDtypeStruct((_SC_ROWS, dim), emb_weight.dtype),
        scratch_types=[
            pltpu.VMEM((_NBUF, _CHUNK, dim), emb_weight.dtype),
            pltpu.SemaphoreType.DMA((2 * _NBUF,)),
        ],
    )(_sc_body)
    sc_out = sc_k(emb_weight[_TC_ROWS:seq_len])

    out = jnp.concatenate([tc_out, sc_out], axis=0)
    return out[None]


# speedup vs baseline: 2.0256x; 2.0256x over previous
import functools

import jax
import jax.numpy as jnp
from jax import lax
from jax.experimental import pallas as pl
from jax.experimental.pallas import tpu as pltpu
from jax.experimental.pallas import tpu_sc as plsc

_NC = 2
_SEQ = 8192
_DIM = 2048
_ROWS_PER_C = _SEQ // _NC
_CHUNK = 256
_NCHUNKS = _ROWS_PER_C // _CHUNK
_NBUF = 2


def _sc_body(w_hbm, out_hbm, buf, sems):
    cid = lax.axis_index("c")
    base = cid * _ROWS_PER_C

    def cp_in(c):
        return pltpu.make_async_copy(
            w_hbm.at[pl.ds(base + c * _CHUNK, _CHUNK), :],
            buf.at[c % _NBUF],
            sems.at[c % _NBUF],
        )

    def cp_out(c):
        return pltpu.make_async_copy(
            buf.at[c % _NBUF],
            out_hbm.at[pl.ds(base + c * _CHUNK, _CHUNK), :],
            sems.at[_NBUF + c % _NBUF],
        )

    for c in range(_NBUF):
        cp_in(c).start()
    for c in range(_NCHUNKS):
        cp_in(c).wait()
        cp_out(c).start()
        if c + _NBUF < _NCHUNKS:
            cp_out(c).wait()
            cp_in(c + _NBUF).start()
    for c in range(max(0, _NCHUNKS - _NBUF), _NCHUNKS):
        cp_out(c).wait()


def kernel(x, emb_weight):
    seq_len = x.shape[1]
    dim = emb_weight.shape[1]
    mesh = plsc.ScalarSubcoreMesh(axis_name="c", num_cores=_NC)
    k = functools.partial(
        pl.kernel,
        mesh=mesh,
        out_type=jax.ShapeDtypeStruct((seq_len, dim), emb_weight.dtype),
        scratch_types=[
            pltpu.VMEM_SHARED((_NBUF, _CHUNK, dim), emb_weight.dtype),
            pltpu.SemaphoreType.DMA((2 * _NBUF,)),
        ],
    )(_sc_body)
    out = k(emb_weight[:seq_len])
    return out[None]


# speedup vs baseline: 2.2103x; 1.0912x over previous
import functools

import jax
import jax.numpy as jnp
from jax import lax
from jax.experimental import pallas as pl
from jax.experimental.pallas import tpu as pltpu
from jax.experimental.pallas import tpu_sc as plsc

_NC, _NS = 2, 16
_NW = _NC * _NS
_SEQ = 8192
_DIM = 2048
_ROWS_PER_W = _SEQ // _NW
_CHUNK = 16
_NCHUNKS = _ROWS_PER_W // _CHUNK
_NBUF = 2


def _sc_body(w_hbm, out_hbm, buf, sems):
    wid = lax.axis_index("s") * _NC + lax.axis_index("c")
    base = wid * _ROWS_PER_W

    def cp_in(c):
        return pltpu.make_async_copy(
            w_hbm.at[pl.ds(base + c * _CHUNK, _CHUNK), :],
            buf.at[c % _NBUF],
            sems.at[c % _NBUF],
        )

    def cp_out(c):
        return pltpu.make_async_copy(
            buf.at[c % _NBUF],
            out_hbm.at[pl.ds(base + c * _CHUNK, _CHUNK), :],
            sems.at[_NBUF + c % _NBUF],
        )

    for c in range(_NBUF):
        cp_in(c).start()
    for c in range(_NCHUNKS):
        cp_in(c).wait()
        cp_out(c).start()
        if c + _NBUF < _NCHUNKS:
            cp_out(c).wait()
            cp_in(c + _NBUF).start()
    for c in range(max(0, _NCHUNKS - _NBUF), _NCHUNKS):
        cp_out(c).wait()


def kernel(x, emb_weight):
    seq_len = x.shape[1]
    dim = emb_weight.shape[1]
    mesh = plsc.VectorSubcoreMesh(core_axis_name="c", subcore_axis_name="s")
    k = functools.partial(
        pl.kernel,
        mesh=mesh,
        out_type=jax.ShapeDtypeStruct((seq_len, dim), emb_weight.dtype),
        scratch_types=[
            pltpu.VMEM((_NBUF, _CHUNK, dim), emb_weight.dtype),
            pltpu.SemaphoreType.DMA((2 * _NBUF,)),
        ],
    )(_sc_body)
    out = k(emb_weight[:seq_len])
    return out[None]
